# Initial kernel scaffold; baseline (speedup 1.0000x reference)
import functools

import jax
import jax.numpy as jnp
from jax import lax
from jax.experimental import pallas as pl
from jax.experimental.pallas import tpu as pltpu
from jax.experimental.pallas import tpu_sc as plsc

N = 10000
E = 320000
D = 128
OUT = 128

NC = 2
NS = 16
NW = NC * NS
EPW = E // NW
C = 80
NCHUNK = EPW // C
ROWS_PER_TILE = N // NS
ZROWS = 125
LANES = 16


def _bcast_lane(vec, i):
  idx = jnp.full((LANES,), i, dtype=jnp.int32)
  return jnp.take(vec, idx, mode="promise_in_bounds")


_sc_mesh = plsc.VectorSubcoreMesh(core_axis_name="c", subcore_axis_name="s")


@functools.partial(
    pl.kernel,
    out_type=jax.ShapeDtypeStruct((NC, N, D), jnp.float32),
    mesh=_sc_mesh,
    scratch_types=[
        pltpu.VMEM((NCHUNK, C), jnp.int32),
        pltpu.VMEM((NCHUNK, C), jnp.int32),
        pltpu.VMEM((NCHUNK, C), jnp.float32),
        pltpu.VMEM((C, D), jnp.float32),
        pltpu.VMEM((ZROWS, D), jnp.float32),
        pltpu.VMEM_SHARED((N, D), jnp.float32),
        pltpu.SemaphoreType.DMA,
    ],
)
def _sc_aggregate(src_hbm, dst_hbm, w_hbm, x_hbm, out_hbm,
                  src_v, dst_v, w_v, rows_v, zbuf_v, acc_sh, sem):
  c = lax.axis_index("c")
  s = lax.axis_index("s")
  wid = s * NC + c

  pltpu.sync_copy(src_hbm.at[wid], src_v)
  pltpu.sync_copy(dst_hbm.at[wid], dst_v)
  pltpu.sync_copy(w_hbm.at[wid], w_v)

  def zero_row(i, _):
    for j in range(D // LANES):
      zbuf_v[i, pl.ds(j * LANES, LANES)] = jnp.zeros((LANES,), jnp.float32)
    return _
  lax.fori_loop(0, ZROWS, zero_row, None)
  for k in range(ROWS_PER_TILE // ZROWS):
    pltpu.sync_copy(
        zbuf_v, acc_sh.at[pl.ds(s * ROWS_PER_TILE + k * ZROWS, ZROWS)])
  plsc.subcore_barrier()

  def chunk_body(ci, _):
    pltpu.async_copy(x_hbm.at[src_v.at[ci]], rows_v, sem).wait()
    for g in range(C // LANES):
      wv = w_v[ci, pl.ds(g * LANES, LANES)]
      for i in range(LANES):
        e = g * LANES + i
        wb = _bcast_lane(wv, i)
        for j in range(D // LANES):
          sl = pl.ds(j * LANES, LANES)
          rows_v[e, sl] = rows_v[e, sl] * wb
    pltpu.sync_copy(rows_v, acc_sh.at[dst_v.at[ci]], add=True)
    return _
  lax.fori_loop(0, NCHUNK, chunk_body, None)
  plsc.subcore_barrier()

  pltpu.sync_copy(acc_sh.at[pl.ds(s * ROWS_PER_TILE, ROWS_PER_TILE)],
                  out_hbm.at[c, pl.ds(s * ROWS_PER_TILE, ROWS_PER_TILE)])


BM = 1000


def _tc_body(p0_ref, p1_ref, x_ref, wt_ref, b_ref, o_ref):
  agg = p0_ref[...] + p1_ref[...]
  sup = jnp.concatenate([agg, x_ref[...]], axis=1)
  acc = jnp.dot(sup, wt_ref[...], preferred_element_type=jnp.float32)
  o_ref[...] = jnp.maximum(acc + b_ref[...], 0.0)


_tc_finish = pl.pallas_call(
    _tc_body,
    grid=(N // BM,),
    in_specs=[
        pl.BlockSpec((BM, D), lambda i: (i, 0)),
        pl.BlockSpec((BM, D), lambda i: (i, 0)),
        pl.BlockSpec((BM, D), lambda i: (i, 0)),
        pl.BlockSpec((2 * D, OUT), lambda i: (0, 0)),
        pl.BlockSpec((1, OUT), lambda i: (0, 0)),
    ],
    out_specs=pl.BlockSpec((BM, OUT), lambda i: (i, 0)),
    out_shape=jax.ShapeDtypeStruct((N, OUT), jnp.float32),
)


def kernel(x, edge_index, edge_weight, W, b):
  src = edge_index[0].reshape(NW, NCHUNK, C)
  dst = edge_index[1].reshape(NW, NCHUNK, C)
  w3 = edge_weight.reshape(NW, NCHUNK, C)
  partials = _sc_aggregate(src, dst, w3, x)
  Wt = W.T
  b2 = b.reshape(1, OUT)
  return _tc_finish(partials[0], partials[1], x, Wt, b2)


# speedup vs baseline: 6.4473x; 6.4473x over previous
import functools

import jax
import jax.numpy as jnp
from jax import lax
from jax.experimental import pallas as pl
from jax.experimental.pallas import tpu as pltpu
from jax.experimental.pallas import tpu_sc as plsc

N = 10000
E = 320000
D = 128
OUT = 128

NC = 2
NS = 16
NW = NC * NS
EPW = E // NW
C = 80
NCHUNK = EPW // C
SEG = 25
NSEG = NCHUNK // SEG
ROWS_MAIN = 624
ROWS_TAIL = N - NS * ROWS_MAIN
ZROWS = 48
LANES = 16


_GATHER_DNUMS = lax.GatherDimensionNumbers(
    offset_dims=(), collapsed_slice_dims=(0,), start_index_map=(0,))


def _bcast_lane(vec, i):
  idx = jnp.full((LANES, 1), i, dtype=jnp.int32)
  return lax.gather(vec, idx, _GATHER_DNUMS, (1,),
                    mode=lax.GatherScatterMode.PROMISE_IN_BOUNDS)


_sc_mesh = plsc.VectorSubcoreMesh(core_axis_name="c", subcore_axis_name="s")


@functools.partial(
    pl.kernel,
    out_type=jax.ShapeDtypeStruct((NC, N, D), jnp.float32),
    mesh=_sc_mesh,
    scratch_types=[
        pltpu.VMEM((SEG, C), jnp.int32),
        pltpu.VMEM((SEG, C), jnp.int32),
        pltpu.VMEM((SEG, C), jnp.float32),
        pltpu.VMEM((C, D), jnp.float32),
        pltpu.VMEM((ZROWS, D), jnp.float32),
        pltpu.VMEM_SHARED((N, D), jnp.float32),
        pltpu.SemaphoreType.DMA,
    ],
)
def _sc_aggregate(src_hbm, dst_hbm, w_hbm, x_hbm, out_hbm,
                  src_v, dst_v, w_v, rows_v, zbuf_v, acc_sh, sem):
  c = lax.axis_index("c")
  s = lax.axis_index("s")
  wid = s * NC + c

  def zero_row(i, _):
    for j in range(D // LANES):
      zbuf_v[i, pl.ds(j * LANES, LANES)] = jnp.zeros((LANES,), jnp.float32)
    return _
  lax.fori_loop(0, ZROWS, zero_row, None)
  for k in range(ROWS_MAIN // ZROWS):
    pltpu.sync_copy(
        zbuf_v, acc_sh.at[pl.ds(s * ROWS_MAIN + k * ZROWS, ZROWS)])
  @pl.when(s == NS - 1)
  def _zero_tail():
    pltpu.sync_copy(zbuf_v.at[pl.ds(0, ROWS_TAIL)],
                    acc_sh.at[pl.ds(NS * ROWS_MAIN, ROWS_TAIL)])
  plsc.subcore_barrier()

  def seg_body(si, _):
    pltpu.sync_copy(src_hbm.at[wid, si], src_v)
    pltpu.sync_copy(dst_hbm.at[wid, si], dst_v)
    pltpu.sync_copy(w_hbm.at[wid, si], w_v)
    def chunk_body(ci, __):
      pltpu.async_copy(x_hbm.at[src_v.at[ci]], rows_v, sem).wait()
      for g in range(C // LANES):
        wv = w_v[ci, pl.ds(g * LANES, LANES)]
        for i in range(LANES):
          e = g * LANES + i
          wb = _bcast_lane(wv, i)
          for j in range(D // LANES):
            sl = pl.ds(j * LANES, LANES)
            rows_v[e, sl] = rows_v[e, sl] * wb
      pltpu.sync_copy(rows_v, acc_sh.at[dst_v.at[ci]], add=True)
      return __
    lax.fori_loop(0, SEG, chunk_body, None)
    return _
  lax.fori_loop(0, NSEG, seg_body, None)
  plsc.subcore_barrier()

  pltpu.sync_copy(acc_sh.at[pl.ds(s * ROWS_MAIN, ROWS_MAIN)],
                  out_hbm.at[c, pl.ds(s * ROWS_MAIN, ROWS_MAIN)])
  @pl.when(s == NS - 1)
  def _pub_tail():
    pltpu.sync_copy(acc_sh.at[pl.ds(NS * ROWS_MAIN, ROWS_TAIL)],
                    out_hbm.at[c, pl.ds(NS * ROWS_MAIN, ROWS_TAIL)])


BM = 1000


def _tc_body(p0_ref, p1_ref, x_ref, wt_ref, b_ref, o_ref):
  agg = p0_ref[...] + p1_ref[...]
  sup = jnp.concatenate([agg, x_ref[...]], axis=1)
  acc = jnp.dot(sup, wt_ref[...], preferred_element_type=jnp.float32)
  o_ref[...] = jnp.maximum(acc + b_ref[...], 0.0)


_tc_finish = pl.pallas_call(
    _tc_body,
    grid=(N // BM,),
    in_specs=[
        pl.BlockSpec((BM, D), lambda i: (i, 0)),
        pl.BlockSpec((BM, D), lambda i: (i, 0)),
        pl.BlockSpec((BM, D), lambda i: (i, 0)),
        pl.BlockSpec((2 * D, OUT), lambda i: (0, 0)),
        pl.BlockSpec((1, OUT), lambda i: (0, 0)),
    ],
    out_specs=pl.BlockSpec((BM, OUT), lambda i: (i, 0)),
    out_shape=jax.ShapeDtypeStruct((N, OUT), jnp.float32),
)


def kernel(x, edge_index, edge_weight, W, b):
  src = edge_index[0].reshape(NW, NSEG, SEG, C)
  dst = edge_index[1].reshape(NW, NSEG, SEG, C)
  w3 = edge_weight.reshape(NW, NSEG, SEG, C)
  partials = _sc_aggregate(src, dst, w3, x)
  Wt = W.T
  b2 = b.reshape(1, OUT)
  return _tc_finish(partials[0], partials[1], x, Wt, b2)


# speedup vs baseline: 9.4993x; 1.4734x over previous
import functools

import jax
import jax.numpy as jnp
from jax import lax
from jax.experimental import pallas as pl
from jax.experimental.pallas import tpu as pltpu
from jax.experimental.pallas import tpu_sc as plsc

N = 10000
E = 320000
D = 128
OUT = 128

NC = 2
NS = 16
NW = NC * NS
EPW = E // NW
C = 80
NCHUNK = EPW // C
SEG = 25
NSEG = NCHUNK // SEG
ROWS_MAIN = 624
ROWS_TAIL = N - NS * ROWS_MAIN
LANES = 16


_GATHER_DNUMS = lax.GatherDimensionNumbers(
    offset_dims=(), collapsed_slice_dims=(0,), start_index_map=(0,))


def _bcast_lane(vec, i):
  idx = jnp.full((LANES, 1), i, dtype=jnp.int32)
  return lax.gather(vec, idx, _GATHER_DNUMS, (1,),
                    mode=lax.GatherScatterMode.PROMISE_IN_BOUNDS)


_sc_mesh = plsc.VectorSubcoreMesh(core_axis_name="c", subcore_axis_name="s")


@functools.partial(
    pl.kernel,
    out_type=jax.ShapeDtypeStruct((NC, N, D), jnp.float32),
    mesh=_sc_mesh,
    scratch_types=[
        pltpu.VMEM((SEG, C), jnp.int32),
        pltpu.VMEM((SEG, C), jnp.int32),
        pltpu.VMEM((SEG, C), jnp.float32),
        pltpu.VMEM((2, C, D), jnp.float32),
        pltpu.VMEM_SHARED((N, D), jnp.float32),
        pltpu.SemaphoreType.DMA,
        pltpu.SemaphoreType.DMA,
    ],
)
def _sc_aggregate(src_hbm, dst_hbm, w_hbm, x_hbm, out_hbm,
                  src_v, dst_v, w_v, rows_v, acc_sh, gsem, ssem):
  c = lax.axis_index("c")
  s = lax.axis_index("s")
  wid = s * NC + c

  def zero_row(i, _):
    for j in range(D // LANES):
      rows_v[0, i, pl.ds(j * LANES, LANES)] = jnp.zeros((LANES,), jnp.float32)
    return _
  lax.fori_loop(0, C, zero_row, None)
  for k in range(ROWS_MAIN // C):
    pltpu.sync_copy(
        rows_v.at[0], acc_sh.at[pl.ds(s * ROWS_MAIN + k * C, C)])
  rem = ROWS_MAIN - (ROWS_MAIN // C) * C
  if rem:
    pltpu.sync_copy(
        rows_v.at[0, pl.ds(0, rem)],
        acc_sh.at[pl.ds(s * ROWS_MAIN + (ROWS_MAIN // C) * C, rem)])
  @pl.when(s == NS - 1)
  def _zero_tail():
    pltpu.sync_copy(rows_v.at[0, pl.ds(0, ROWS_TAIL)],
                    acc_sh.at[pl.ds(NS * ROWS_MAIN, ROWS_TAIL)])
  plsc.subcore_barrier()

  def seg_body(si, _):
    pltpu.sync_copy(src_hbm.at[wid, si], src_v)
    pltpu.sync_copy(dst_hbm.at[wid, si], dst_v)
    pltpu.sync_copy(w_hbm.at[wid, si], w_v)
    pltpu.async_copy(x_hbm.at[src_v.at[0]], rows_v.at[0], gsem)
    def chunk_body(ci, __):
      buf = lax.rem(ci, 2)
      obuf = 1 - buf
      pltpu.make_async_copy(x_hbm.at[src_v.at[ci]], rows_v.at[buf],
                            gsem).wait()
      @pl.when(ci >= 1)
      def _wait_prev_scatter():
        pltpu.make_async_copy(rows_v.at[obuf],
                              acc_sh.at[dst_v.at[ci - 1]], ssem).wait()
      @pl.when(ci + 1 < SEG)
      def _prefetch_next():
        pltpu.async_copy(x_hbm.at[src_v.at[ci + 1]], rows_v.at[obuf], gsem)
      for g in range(C // LANES):
        wv = w_v[ci, pl.ds(g * LANES, LANES)]
        for i in range(LANES):
          e = g * LANES + i
          wb = _bcast_lane(wv, i)
          for j in range(D // LANES):
            sl = pl.ds(j * LANES, LANES)
            rows_v[buf, e, sl] = rows_v[buf, e, sl] * wb
      pltpu.async_copy(rows_v.at[buf], acc_sh.at[dst_v.at[ci]], ssem,
                       add=True)
      return __
    lax.fori_loop(0, SEG, chunk_body, None)
    pltpu.make_async_copy(rows_v.at[(SEG - 1) % 2],
                          acc_sh.at[dst_v.at[SEG - 1]], ssem).wait()
    return _
  lax.fori_loop(0, NSEG, seg_body, None)
  plsc.subcore_barrier()

  pltpu.sync_copy(acc_sh.at[pl.ds(s * ROWS_MAIN, ROWS_MAIN)],
                  out_hbm.at[c, pl.ds(s * ROWS_MAIN, ROWS_MAIN)])
  @pl.when(s == NS - 1)
  def _pub_tail():
    pltpu.sync_copy(acc_sh.at[pl.ds(NS * ROWS_MAIN, ROWS_TAIL)],
                    out_hbm.at[c, pl.ds(NS * ROWS_MAIN, ROWS_TAIL)])


BM = 1000


def _tc_body(p_ref, x_ref, wt_ref, b_ref, o_ref):
  agg = p_ref[0] + p_ref[1]
  sup = jnp.concatenate([agg, x_ref[...]], axis=1)
  acc = jnp.dot(sup, wt_ref[...], preferred_element_type=jnp.float32)
  o_ref[...] = jnp.maximum(acc + b_ref[...], 0.0)


_tc_finish = pl.pallas_call(
    _tc_body,
    grid=(N // BM,),
    in_specs=[
        pl.BlockSpec((2, BM, D), lambda i: (0, i, 0)),
        pl.BlockSpec((BM, D), lambda i: (i, 0)),
        pl.BlockSpec((2 * D, OUT), lambda i: (0, 0)),
        pl.BlockSpec((1, OUT), lambda i: (0, 0)),
    ],
    out_specs=pl.BlockSpec((BM, OUT), lambda i: (i, 0)),
    out_shape=jax.ShapeDtypeStruct((N, OUT), jnp.float32),
)


def kernel(x, edge_index, edge_weight, W, b):
  src = edge_index[0].reshape(NW, NSEG, SEG, C)
  dst = edge_index[1].reshape(NW, NSEG, SEG, C)
  w3 = edge_weight.reshape(NW, NSEG, SEG, C)
  partials = _sc_aggregate(src, dst, w3, x)
  Wt = W.T
  b2 = b.reshape(1, OUT)
  return _tc_finish(partials, x, Wt, b2)


# speedup vs baseline: 9.6329x; 1.0141x over previous
import functools

import jax
import jax.numpy as jnp
from jax import lax
from jax.experimental import pallas as pl
from jax.experimental.pallas import tpu as pltpu
from jax.experimental.pallas import tpu_sc as plsc

N = 10000
E = 320000
D = 128
OUT = 128

NC = 2
NS = 16
NW = NC * NS
EPW = E // NW
C = 80
NCHUNK = EPW // C
SEG = 25
NSEG = NCHUNK // SEG
ROWS_MAIN = 624
ROWS_TAIL = N - NS * ROWS_MAIN
LANES = 16


_GATHER_DNUMS = lax.GatherDimensionNumbers(
    offset_dims=(), collapsed_slice_dims=(0,), start_index_map=(0,))


def _bcast_lane(vec, i):
  idx = jnp.full((LANES, 1), i, dtype=jnp.int32)
  return lax.gather(vec, idx, _GATHER_DNUMS, (1,),
                    mode=lax.GatherScatterMode.PROMISE_IN_BOUNDS)


_sc_mesh = plsc.VectorSubcoreMesh(core_axis_name="c", subcore_axis_name="s")


@functools.partial(
    pl.kernel,
    out_type=jax.ShapeDtypeStruct((NC, N, D), jnp.float32),
    mesh=_sc_mesh,
    scratch_types=[
        pltpu.VMEM((SEG * C,), jnp.int32),
        pltpu.VMEM((SEG, C), jnp.int32),
        pltpu.VMEM((SEG * C,), jnp.float32),
        pltpu.VMEM((2, C, D), jnp.float32),
        pltpu.VMEM_SHARED((N, D), jnp.float32),
        pltpu.SemaphoreType.DMA,
        pltpu.SemaphoreType.DMA,
    ],
)
def _sc_aggregate(src_hbm, dst_hbm, w_hbm, x_hbm, out_hbm,
                  src_v, dst_v, w_v, rows_v, acc_sh, gsem, ssem):
  c = lax.axis_index("c")
  s = lax.axis_index("s")
  wid = s * NC + c

  def zero_row(i, _):
    for j in range(D // LANES):
      rows_v[0, i, pl.ds(j * LANES, LANES)] = jnp.zeros((LANES,), jnp.float32)
    return _
  lax.fori_loop(0, C, zero_row, None)
  for k in range(ROWS_MAIN // C):
    pltpu.sync_copy(
        rows_v.at[0], acc_sh.at[pl.ds(s * ROWS_MAIN + k * C, C)])
  rem = ROWS_MAIN - (ROWS_MAIN // C) * C
  if rem:
    pltpu.sync_copy(
        rows_v.at[0, pl.ds(0, rem)],
        acc_sh.at[pl.ds(s * ROWS_MAIN + (ROWS_MAIN // C) * C, rem)])
  @pl.when(s == NS - 1)
  def _zero_tail():
    pltpu.sync_copy(rows_v.at[0, pl.ds(0, ROWS_TAIL)],
                    acc_sh.at[pl.ds(NS * ROWS_MAIN, ROWS_TAIL)])
  plsc.subcore_barrier()

  def seg_body(si, _):
    seg_base = wid * EPW + si * (SEG * C)
    pltpu.sync_copy(src_hbm.at[pl.ds(seg_base, SEG * C)], src_v)
    pltpu.sync_copy(dst_hbm.at[wid, si], dst_v)
    pltpu.sync_copy(w_hbm.at[pl.ds(seg_base, SEG * C)], w_v)
    pltpu.async_copy(x_hbm.at[src_v.at[pl.ds(0, C)]], rows_v.at[0], gsem)
    def chunk_body(ci, __):
      buf = lax.rem(ci, 2)
      obuf = 1 - buf
      pltpu.make_async_copy(x_hbm.at[src_v.at[pl.ds(ci * C, C)]],
                            rows_v.at[buf], gsem).wait()
      @pl.when(ci >= 1)
      def _wait_prev_scatter():
        pltpu.make_async_copy(rows_v.at[obuf],
                              acc_sh.at[dst_v.at[ci - 1]], ssem).wait()
      @pl.when(ci + 1 < SEG)
      def _prefetch_next():
        pltpu.async_copy(x_hbm.at[src_v.at[pl.ds((ci + 1) * C, C)]],
                         rows_v.at[obuf], gsem)
      for g in range(C // LANES):
        wv = w_v[pl.ds(ci * C + g * LANES, LANES)]
        for i in range(LANES):
          e = g * LANES + i
          wb = _bcast_lane(wv, i)
          for j in range(D // LANES):
            sl = pl.ds(j * LANES, LANES)
            rows_v[buf, e, sl] = rows_v[buf, e, sl] * wb
      pltpu.async_copy(rows_v.at[buf], acc_sh.at[dst_v.at[ci]], ssem,
                       add=True)
      return __
    lax.fori_loop(0, SEG, chunk_body, None)
    pltpu.make_async_copy(rows_v.at[(SEG - 1) % 2],
                          acc_sh.at[dst_v.at[SEG - 1]], ssem).wait()
    return _
  lax.fori_loop(0, NSEG, seg_body, None)
  plsc.subcore_barrier()

  pltpu.sync_copy(acc_sh.at[pl.ds(s * ROWS_MAIN, ROWS_MAIN)],
                  out_hbm.at[c, pl.ds(s * ROWS_MAIN, ROWS_MAIN)])
  @pl.when(s == NS - 1)
  def _pub_tail():
    pltpu.sync_copy(acc_sh.at[pl.ds(NS * ROWS_MAIN, ROWS_TAIL)],
                    out_hbm.at[c, pl.ds(NS * ROWS_MAIN, ROWS_TAIL)])


BM = 1000


def _tc_body(p_ref, x_ref, wt_ref, b_ref, o_ref):
  agg = p_ref[0] + p_ref[1]
  sup = jnp.concatenate([agg, x_ref[...]], axis=1)
  acc = jnp.dot(sup, wt_ref[...], preferred_element_type=jnp.float32)
  o_ref[...] = jnp.maximum(acc + b_ref[...], 0.0)


_tc_finish = pl.pallas_call(
    _tc_body,
    grid=(N // BM,),
    in_specs=[
        pl.BlockSpec((2, BM, D), lambda i: (0, i, 0)),
        pl.BlockSpec((BM, D), lambda i: (i, 0)),
        pl.BlockSpec((2 * D, OUT), lambda i: (0, 0)),
        pl.BlockSpec((1, OUT), lambda i: (0, 0)),
    ],
    out_specs=pl.BlockSpec((BM, OUT), lambda i: (i, 0)),
    out_shape=jax.ShapeDtypeStruct((N, OUT), jnp.float32),
)


def kernel(x, edge_index, edge_weight, W, b):
  dst = edge_index[1].reshape(NW, NSEG, SEG, C)
  partials = _sc_aggregate(edge_index[0], dst, edge_weight, x)
  Wt = W.T
  b2 = b.reshape(1, OUT)
  return _tc_finish(partials, x, Wt, b2)
